# baseline (speedup 1.0000x reference)
import functools
import math

import jax
import jax.numpy as jnp
import numpy as np
from jax.experimental import pallas as pl
from jax.experimental.pallas import tpu as pltpu

H = 96
QK = 8
VB = 8
W = 8
HALO = 128
T = 1024

_INV_SQRT_QK = 1.0 / math.sqrt(float(QK))
_STD_OF_BM = np.arange(H * QK).reshape(H, QK).T.reshape(-1)
_GT_BM = np.tile(np.eye(H, dtype=np.float32), (1, QK))

_NT = (((1,), (1,)), ((), ()))
_TN = (((0,), (0,)), ((), ()))


def _rosa_body(h_ref, wq_ref, wk_ref, wv_ref, wo_ref, bias_ref, gt_ref,
               out_ref, kext_ref, vext_ref):
    i = pl.program_id(0)
    h = h_ref[...].astype(jnp.bfloat16)
    q = jax.lax.dot_general(wq_ref[...], h, _NT,
                            preferred_element_type=jnp.float32)
    k = jax.lax.dot_general(wk_ref[...], h, _NT,
                            preferred_element_type=jnp.float32)
    v = jax.lax.dot_general(wv_ref[...], h, _NT,
                            preferred_element_type=jnp.float32)
    qb = jnp.tanh(q).astype(jnp.bfloat16)
    kb = jnp.tanh(k).astype(jnp.bfloat16)
    vb = jax.nn.sigmoid(v).astype(jnp.bfloat16)

    @pl.when(i == 0)
    def _init_halo():
        kext_ref[:, 0:HALO] = jnp.zeros((H * QK, HALO), jnp.bfloat16)
        vext_ref[:, 0:HALO] = jnp.zeros((H * VB, HALO), jnp.bfloat16)

    @pl.when(i > 0)
    def _carry_halo():
        kext_ref[:, 0:HALO] = kext_ref[:, T:T + HALO]
        vext_ref[:, 0:HALO] = vext_ref[:, T:T + HALO]

    kext_ref[:, HALO:] = kb
    vext_ref[:, HALO:] = vb

    pos = i * T + jax.lax.broadcasted_iota(jnp.int32, (H, T), 1)
    scores = []
    for o in range(W):
        prod = qb * (kb if o == 0 else kext_ref[:, HALO - o:HALO - o + T])
        s = jnp.dot(gt_ref[...], prod,
                    preferred_element_type=jnp.float32) * _INV_SQRT_QK
        if o > 0:
            s = jnp.where(pos >= o, s, -1e30)
        scores.append(s)
    m = functools.reduce(jnp.maximum, scores)
    exps = [jnp.exp(s - m) for s in scores]
    inv = 1.0 / functools.reduce(lambda a, b: a + b, exps)
    acc = None
    for o in range(W):
        p = (exps[o] * inv).astype(jnp.bfloat16)
        pw = jnp.concatenate([p] * VB, axis=0)
        term = pw * (vb if o == 0 else vext_ref[:, HALO - o:HALO - o + T])
        acc = term if acc is None else acc + term
    res = jax.lax.dot_general(acc, wo_ref[...], _TN,
                              preferred_element_type=jnp.float32)
    out_ref[...] = res + bias_ref[...]


def kernel(hidden_states, Wq, Wk, Wv, Wo, v_emb0, v_emb1):
    b, s, hid = hidden_states.shape
    h2 = hidden_states.reshape(b * s, hid)
    perm = _STD_OF_BM
    wq = Wq[perm].astype(jnp.bfloat16)
    wk = Wk[perm].astype(jnp.bfloat16)
    wv = Wv[perm].astype(jnp.bfloat16)
    wo = (((v_emb1 - v_emb0)[:, None] * Wo.T)[perm]).astype(jnp.bfloat16)
    bias = (Wo @ v_emb0).reshape(1, hid)
    gt = jnp.asarray(_GT_BM, dtype=jnp.bfloat16)
    nb = (b * s) // T

    out = pl.pallas_call(
        _rosa_body,
        grid=(nb,),
        in_specs=[
            pl.BlockSpec((T, hid), lambda i: (i, 0)),
            pl.BlockSpec((H * QK, hid), lambda i: (0, 0)),
            pl.BlockSpec((H * QK, hid), lambda i: (0, 0)),
            pl.BlockSpec((H * VB, hid), lambda i: (0, 0)),
            pl.BlockSpec((H * VB, hid), lambda i: (0, 0)),
            pl.BlockSpec((1, hid), lambda i: (0, 0)),
            pl.BlockSpec((H, H * QK), lambda i: (0, 0)),
        ],
        out_specs=pl.BlockSpec((T, hid), lambda i: (i, 0)),
        out_shape=jax.ShapeDtypeStruct((b * s, hid), jnp.float32),
        scratch_shapes=[
            pltpu.VMEM((H * QK, T + HALO), jnp.bfloat16),
            pltpu.VMEM((H * VB, T + HALO), jnp.bfloat16),
        ],
    )(h2, wq, wk, wv, wo, bias, gt)
    return out.reshape(b, s, hid)


# speedup vs baseline: 1.1055x; 1.1055x over previous
import functools
import math

import jax
import jax.numpy as jnp
import numpy as np
from jax.experimental import pallas as pl
from jax.experimental.pallas import tpu as pltpu

H = 96
QK = 8
VB = 8
W = 8
HALO = 16
T = 1024

_INV_SQRT_QK = 1.0 / math.sqrt(float(QK))
_STD_OF_BM = np.arange(H * QK).reshape(H, QK).T.reshape(-1)
_G_BM = np.tile(np.eye(H, dtype=np.float32), (QK, 1))

_NT = (((1,), (1,)), ((), ()))


def _rosa_body(h_ref, wq_ref, wk_ref, wv_ref, wo_ref, bias_ref, g_ref,
               demb_ref, out_ref, kext_ref, vext_ref):
    i = pl.program_id(0)
    h = h_ref[...].astype(jnp.bfloat16)
    q = jax.lax.dot_general(h, wq_ref[...], _NT,
                            preferred_element_type=jnp.float32)
    k = jax.lax.dot_general(h, wk_ref[...], _NT,
                            preferred_element_type=jnp.float32)
    v = jax.lax.dot_general(h, wv_ref[...], _NT,
                            preferred_element_type=jnp.float32)
    qb = jnp.tanh(q).astype(jnp.bfloat16)
    kb = jnp.tanh(k).astype(jnp.bfloat16)
    vb = (jax.nn.sigmoid(v) * demb_ref[...]).astype(jnp.bfloat16)

    @pl.when(i == 0)
    def _init_halo():
        kext_ref[0:HALO] = jnp.zeros((HALO, H * QK), jnp.bfloat16)
        vext_ref[0:HALO] = jnp.zeros((HALO, H * VB), jnp.bfloat16)

    @pl.when(i > 0)
    def _carry_halo():
        kext_ref[0:HALO] = kext_ref[T:T + HALO]
        vext_ref[0:HALO] = vext_ref[T:T + HALO]

    kext_ref[HALO:] = kb
    vext_ref[HALO:] = vb

    row = i * T + jax.lax.broadcasted_iota(jnp.int32, (T, H), 0)
    scores = []
    for o in range(W):
        prod = qb * (kb if o == 0 else kext_ref[HALO - o:HALO - o + T])
        s = jnp.dot(prod, g_ref[...],
                    preferred_element_type=jnp.float32) * _INV_SQRT_QK
        if o > 0:
            s = jnp.where(row >= o, s, -1e30)
        scores.append(s)
    m = functools.reduce(jnp.maximum, scores)
    exps = [jnp.exp(s - m) for s in scores]
    inv = 1.0 / functools.reduce(lambda a, b: a + b, exps)
    acc = None
    for o in range(W):
        p = (exps[o] * inv).astype(jnp.bfloat16)
        pw = jnp.concatenate([p] * VB, axis=1)
        term = pw * (vb if o == 0 else vext_ref[HALO - o:HALO - o + T])
        acc = term if acc is None else acc + term
    res = jax.lax.dot_general(acc, wo_ref[...], _NT,
                              preferred_element_type=jnp.float32)
    out_ref[...] = res + bias_ref[...]


def kernel(hidden_states, Wq, Wk, Wv, Wo, v_emb0, v_emb1):
    b, s, hid = hidden_states.shape
    h2 = hidden_states.reshape(b * s, hid)
    perm = _STD_OF_BM
    wq = Wq[perm].astype(jnp.bfloat16)
    wk = Wk[perm].astype(jnp.bfloat16)
    wv = Wv[perm].astype(jnp.bfloat16)
    wo = Wo[:, perm].astype(jnp.bfloat16)
    demb = (v_emb1 - v_emb0)[perm].reshape(1, H * VB)
    bias = (Wo @ v_emb0).reshape(1, hid)
    g = jnp.asarray(_G_BM, dtype=jnp.bfloat16)
    nb = (b * s) // T

    out = pl.pallas_call(
        _rosa_body,
        grid=(nb,),
        in_specs=[
            pl.BlockSpec((T, hid), lambda i: (i, 0)),
            pl.BlockSpec((H * QK, hid), lambda i: (0, 0)),
            pl.BlockSpec((H * QK, hid), lambda i: (0, 0)),
            pl.BlockSpec((H * VB, hid), lambda i: (0, 0)),
            pl.BlockSpec((hid, H * VB), lambda i: (0, 0)),
            pl.BlockSpec((1, hid), lambda i: (0, 0)),
            pl.BlockSpec((H * QK, H), lambda i: (0, 0)),
            pl.BlockSpec((1, H * VB), lambda i: (0, 0)),
        ],
        out_specs=pl.BlockSpec((T, hid), lambda i: (i, 0)),
        out_shape=jax.ShapeDtypeStruct((b * s, hid), jnp.float32),
        scratch_shapes=[
            pltpu.VMEM((T + HALO, H * QK), jnp.bfloat16),
            pltpu.VMEM((T + HALO, H * VB), jnp.bfloat16),
        ],
    )(h2, wq, wk, wv, wo, bias, g, demb)
    return out.reshape(b, s, hid)


# speedup vs baseline: 1.1669x; 1.0556x over previous
import functools
import math

import jax
import jax.numpy as jnp
import numpy as np
from jax.experimental import pallas as pl
from jax.experimental.pallas import tpu as pltpu

H = 96
QK = 8
VB = 8
W = 8
HALO = 16
T = 1024

_INV_SQRT_QK = 1.0 / math.sqrt(float(QK))
_STD_OF_BM = np.arange(H * QK).reshape(H, QK).T.reshape(-1)
_G_BM = np.tile(np.eye(H, dtype=np.float32), (QK, 1))

_NT = (((1,), (1,)), ((), ()))


def _rosa_body(h_ref, wq_ref, wk_ref, wv_ref, wo_ref, bias_ref, g_ref,
               demb_ref, out_ref, kext_ref, vext_ref):
    i = pl.program_id(0)
    h = h_ref[...].astype(jnp.bfloat16)
    q = jax.lax.dot_general(h, wq_ref[...], _NT,
                            preferred_element_type=jnp.float32)
    k = jax.lax.dot_general(h, wk_ref[...], _NT,
                            preferred_element_type=jnp.float32)
    v = jax.lax.dot_general(h, wv_ref[...], _NT,
                            preferred_element_type=jnp.float32)
    qb = jnp.tanh(q).astype(jnp.bfloat16)
    kb = jnp.tanh(k).astype(jnp.bfloat16)
    vb = (jax.nn.sigmoid(v) * demb_ref[...]).astype(jnp.bfloat16)

    @pl.when(i == 0)
    def _init_halo():
        kext_ref[0:HALO] = jnp.zeros((HALO, H * QK), jnp.bfloat16)
        vext_ref[0:HALO] = jnp.zeros((HALO, H * VB), jnp.bfloat16)

    @pl.when(i > 0)
    def _carry_halo():
        kext_ref[0:HALO] = kext_ref[T:T + HALO]
        vext_ref[0:HALO] = vext_ref[T:T + HALO]

    kext_ref[HALO:] = kb
    vext_ref[HALO:] = vb

    row = i * T + jax.lax.broadcasted_iota(jnp.int32, (T, H), 0)
    scores = []
    for o in range(W):
        prod = qb * (kb if o == 0 else kext_ref[HALO - o:HALO - o + T])
        s = jnp.dot(prod, g_ref[...],
                    preferred_element_type=jnp.float32) * _INV_SQRT_QK
        if o > 0:
            s = jnp.where(row >= o, s, -1e30)
        scores.append(s)
    m = functools.reduce(jnp.maximum, scores)
    exps = [jnp.exp(s - m) for s in scores]
    inv = 1.0 / functools.reduce(lambda a, b: a + b, exps)
    acc = None
    for o in range(W):
        p = (exps[o] * inv).astype(jnp.bfloat16)
        pw = jnp.concatenate([p] * VB, axis=1)
        term = pw * (vb if o == 0 else vext_ref[HALO - o:HALO - o + T])
        acc = term if acc is None else acc + term
    res = jax.lax.dot_general(acc, wo_ref[...], _NT,
                              preferred_element_type=jnp.float32)
    out_ref[...] = res + bias_ref[...]


def kernel(hidden_states, Wq, Wk, Wv, Wo, v_emb0, v_emb1):
    b, s, hid = hidden_states.shape
    h2 = hidden_states.reshape(b * s, hid)
    perm = _STD_OF_BM

    def _perm_rows(w):
        wb = w.astype(jnp.bfloat16)
        return wb.reshape(H, QK, hid).swapaxes(0, 1).reshape(H * QK, hid)

    wq = _perm_rows(Wq)
    wk = _perm_rows(Wk)
    wv = _perm_rows(Wv)
    wo = (Wo.astype(jnp.bfloat16)
          .reshape(hid, H, VB).swapaxes(1, 2).reshape(hid, H * VB))
    demb = (v_emb1 - v_emb0)[perm].reshape(1, H * VB)
    bias = (Wo @ v_emb0).reshape(1, hid)
    g = jnp.asarray(_G_BM, dtype=jnp.bfloat16)
    nb = (b * s) // T

    out = pl.pallas_call(
        _rosa_body,
        grid=(nb,),
        in_specs=[
            pl.BlockSpec((T, hid), lambda i: (i, 0)),
            pl.BlockSpec((H * QK, hid), lambda i: (0, 0)),
            pl.BlockSpec((H * QK, hid), lambda i: (0, 0)),
            pl.BlockSpec((H * VB, hid), lambda i: (0, 0)),
            pl.BlockSpec((hid, H * VB), lambda i: (0, 0)),
            pl.BlockSpec((1, hid), lambda i: (0, 0)),
            pl.BlockSpec((H * QK, H), lambda i: (0, 0)),
            pl.BlockSpec((1, H * VB), lambda i: (0, 0)),
        ],
        out_specs=pl.BlockSpec((T, hid), lambda i: (i, 0)),
        out_shape=jax.ShapeDtypeStruct((b * s, hid), jnp.float32),
        scratch_shapes=[
            pltpu.VMEM((T + HALO, H * QK), jnp.bfloat16),
            pltpu.VMEM((T + HALO, H * VB), jnp.bfloat16),
        ],
    )(h2, wq, wk, wv, wo, bias, g, demb)
    return out.reshape(b, s, hid)
